# Initial kernel scaffold; baseline (speedup 1.0000x reference)
import jax
import jax.numpy as jnp
from jax.experimental import pallas as pl


def kernel(target, context, negative_samples, target_weight, context_weight):
    raise NotImplementedError("write your pallas kernel here")


# speedup vs baseline: 4.0819x; 4.0819x over previous
import jax
import jax.numpy as jnp
from jax.experimental import pallas as pl
from jax.experimental.pallas import tpu as pltpu
from jax.experimental.pallas import tpu_sc as plsc

_GW = 128


def _gather_embeddings(target_weight, context_weight, tgt_idx, ctx_idx, neg_idx):
    D = target_weight.shape[1]
    Bt = tgt_idx.shape[1]
    Bc = ctx_idx.shape[1]
    Bn = neg_idx.shape[1]
    mesh = plsc.VectorSubcoreMesh(core_axis_name="core", subcore_axis_name="subcore")

    @pl.kernel(
        out_type=(
            jax.ShapeDtypeStruct((Bt, D), target_weight.dtype),
            jax.ShapeDtypeStruct((Bc, D), context_weight.dtype),
            jax.ShapeDtypeStruct((Bn, D), context_weight.dtype),
        ),
        mesh=mesh,
        compiler_params=pltpu.CompilerParams(use_tc_tiling_on_sc=False),
    )
    def k(twt_hbm, cwt_hbm, ti_hbm, ci_hbm, ni_hbm, t_out, c_out, n_out):
        def run(table_hbm, idx_hbm, out_hbm, n_rows):
            def body(i_vmem, o_vmem):
                pltpu.sync_copy(table_hbm.at[i_vmem.at[0]], o_vmem)

            pltpu.emit_pipeline(
                body,
                grid=(n_rows // _GW,),
                in_specs=[pl.BlockSpec((1, _GW), index_map=lambda i: (0, i))],
                out_specs=[pl.BlockSpec((_GW, D), index_map=lambda i: (i, 0))],
                core_axis_name=("core", "subcore"),
                dimension_semantics=(pltpu.PARALLEL,),
            )(idx_hbm, out_hbm)

        run(twt_hbm, ti_hbm, t_out, Bt)
        run(cwt_hbm, ci_hbm, c_out, Bc)
        run(cwt_hbm, ni_hbm, n_out, Bn)

    return k(target_weight, context_weight, tgt_idx, ctx_idx, neg_idx)


def _loss_from_embeddings(t_emb, c_emb, n_emb):
    B, D = t_emb.shape
    K = n_emb.shape[1]
    BB = 512

    def body(t_ref, c_ref, n_ref, o_ref):
        i = pl.program_id(0)
        t = t_ref[...]
        c = c_ref[...]
        n = n_ref[...]
        pos = jnp.sum(t * c, axis=1)
        neg = jnp.sum(n * t[:, None, :], axis=2)
        part = (-jnp.sum(jax.nn.log_sigmoid(pos))
                - jnp.sum(jax.nn.log_sigmoid(-neg)))

        @pl.when(i == 0)
        def _():
            o_ref[...] = jnp.zeros_like(o_ref)

        o_ref[...] += part[None, None]

    res = pl.pallas_call(
        body,
        grid=(B // BB,),
        in_specs=[
            pl.BlockSpec((BB, D), lambda i: (i, 0)),
            pl.BlockSpec((BB, D), lambda i: (i, 0)),
            pl.BlockSpec((BB, K, D), lambda i: (i, 0, 0)),
        ],
        out_specs=pl.BlockSpec((1, 1), lambda i: (0, 0)),
        out_shape=jax.ShapeDtypeStruct((1, 1), jnp.float32),
    )(t_emb, c_emb, n_emb)
    return res[0, 0]


def kernel(target, context, negative_samples, target_weight, context_weight):
    B = target.shape[0]
    K = negative_samples.shape[1]
    D = target_weight.shape[1]
    tgt_idx = target.astype(jnp.int32).reshape(1, B)
    ctx_idx = context.astype(jnp.int32).reshape(1, B)
    neg_idx = negative_samples.astype(jnp.int32).reshape(1, B * K)
    t_emb, c_emb, n_emb = _gather_embeddings(
        target_weight, context_weight, tgt_idx, ctx_idx, neg_idx)
    n_emb = n_emb.reshape(B, K, D)
    return _loss_from_embeddings(t_emb, c_emb, n_emb) / B
